# baseline (speedup 1.0000x reference)
import functools

import jax
import jax.numpy as jnp
from jax import lax
from jax.experimental import pallas as pl
from jax.experimental.pallas import tpu as pltpu
from jax.experimental.pallas import tpu_sc as plsc

N = 10000
E = 320000
D = 128
DE = 16
EH = 128
OE = 16
NH = 128
ON = 128

NC = 2
NS = 16
NW = NC * NS

WROWS = 81
EPAD = NW * WROWS * 128

NPAD = 10240
STRIPE = NPAD // NS

BE = 4096
BN = 2000

_SQRT_HALF = 0.7071067811865476


def _ln_gelu(x, g, b):
    mu = jnp.mean(x, axis=-1, keepdims=True)
    xc = x - mu
    var = jnp.mean(xc * xc, axis=-1, keepdims=True)
    y = xc * lax.rsqrt(var + 1e-5) * g + b
    return 0.5 * y * (1.0 + lax.erf(y * _SQRT_HALF))


def _sc_gather(x, row2d, col2d):
    mesh = plsc.VectorSubcoreMesh(core_axis_name="c", subcore_axis_name="s", num_cores=NC, num_subcores=NS)
    NBUF = 6
    LOOKBACK = 3
    NJOBS = 2 * WROWS

    @functools.partial(
        pl.kernel,
        out_type=[
            jax.ShapeDtypeStruct((EPAD, D), jnp.float32),
            jax.ShapeDtypeStruct((EPAD, D), jnp.float32),
        ],
        mesh=mesh,
        scratch_types=[
            pltpu.VMEM((WROWS, 128), jnp.int32),
            pltpu.VMEM((WROWS, 128), jnp.int32),
            pltpu.VMEM((NBUF * 128, D), jnp.float32),
        ] + [pltpu.SemaphoreType.DMA] * (2 * NBUF),
        compiler_params=pltpu.CompilerParams(use_tc_tiling_on_sc=False),
    )
    def gather_k(x_hbm, row_hbm, col_hbm, gs_hbm, gt_hbm,
                 ridx_v, cidx_v, buf_v, *sems):
        gsem = sems[:NBUF]
        wsem = sems[NBUF:]
        wid = lax.axis_index("s") * NC + lax.axis_index("c")
        base = wid * WROWS

        pltpu.sync_copy(row_hbm.at[pl.ds(base, WROWS)], ridx_v)
        pltpu.sync_copy(col_hbm.at[pl.ds(base, WROWS)], cidx_v)

        def gather_cp(chunk, u, is_col):
            idx = (cidx_v if is_col else ridx_v).at[chunk]
            return pltpu.make_async_copy(
                x_hbm.at[idx], buf_v.at[pl.ds(u * 128, 128)], gsem[u])

        def write_cp(chunk, u, is_col):
            dst = (gt_hbm if is_col else gs_hbm).at[pl.ds((base + chunk) * 128, 128)]
            return pltpu.make_async_copy(buf_v.at[pl.ds(u * 128, 128)], dst,
                                         wsem[u])

        def step(jj, u):
            chunk = 3 * jj + u // 2
            is_col = (u % 2) == 1

            def start_gather():
                gather_cp(chunk, u, is_col).start()

            @pl.when(jj >= 1)
            def _():
                write_cp(chunk - 3, u, is_col).wait()
                start_gather()

            @pl.when(jj == 0)
            def _():
                start_gather()

            u2 = (u - LOOKBACK) % NBUF
            jj2 = jj if u >= LOOKBACK else jj - 1
            chunk2 = 3 * jj2 + u2 // 2
            is_col2 = (u2 % 2) == 1

            def drain_and_write():
                gather_cp(chunk2, u2, is_col2).wait()
                write_cp(chunk2, u2, is_col2).start()

            if u >= LOOKBACK:
                drain_and_write()
            else:
                @pl.when(jj >= 1)
                def _():
                    drain_and_write()

        def body(jj, carry):
            for u in range(NBUF):
                step(jj, u)
            return carry

        lax.fori_loop(0, NJOBS // NBUF, body, 0)

        last_jj = NJOBS // NBUF - 1
        for u in range(NBUF - LOOKBACK, NBUF):
            chunk2 = 3 * last_jj + u // 2
            is_col2 = (u % 2) == 1
            gather_cp(chunk2, u, is_col2).wait()
            write_cp(chunk2, u, is_col2).start()
        for u in range(NBUF):
            chunk2 = 3 * last_jj + u // 2
            write_cp(chunk2, u, (u % 2) == 1).wait()

    return gather_k(x, row2d, col2d)


def _sc_segment_sum(eout_p, col2d):
    mesh = plsc.VectorSubcoreMesh(core_axis_name="c", subcore_axis_name="s", num_cores=NC, num_subcores=NS)
    CR = 9

    @functools.partial(
        pl.kernel,
        out_type=jax.ShapeDtypeStruct((NC, NPAD, OE), jnp.float32),
        mesh=mesh,
        scratch_types=[
            pltpu.VMEM((CR, 128), jnp.int32),
            pltpu.VMEM((CR * 128, OE), jnp.float32),
            pltpu.VMEM((STRIPE, OE), jnp.float32),
            pltpu.VMEM_SHARED((NPAD, OE), jnp.float32),
        ],
        compiler_params=pltpu.CompilerParams(use_tc_tiling_on_sc=False),
    )
    def scatter_k(eout_hbm, col_hbm, out_hbm, idx_v, rows_v, zb_v, acc_sp):
        cid = lax.axis_index("c")
        sid = lax.axis_index("s")
        wid = sid * NC + cid

        def zbody(i, carry):
            zb_v[i] = jnp.zeros((OE,), jnp.float32)
            return carry

        lax.fori_loop(0, STRIPE, zbody, 0)
        pltpu.sync_copy(zb_v, acc_sp.at[pl.ds(sid * STRIPE, STRIPE)])
        plsc.subcore_barrier()

        base_row = wid * WROWS

        def body(j, carry):
            r0 = base_row + j * CR
            pltpu.sync_copy(col_hbm.at[pl.ds(r0, CR)], idx_v)
            pltpu.sync_copy(eout_hbm.at[pl.ds(r0 * 128, CR * 128)], rows_v)

            def inner(k, c2):
                pltpu.sync_copy(rows_v.at[pl.ds(k * 128, 128)],
                                acc_sp.at[idx_v.at[k]], add=True)
                return c2

            lax.fori_loop(0, CR, inner, 0)
            return carry

        lax.fori_loop(0, WROWS // CR, body, 0)
        plsc.subcore_barrier()

        pltpu.sync_copy(acc_sp.at[pl.ds(sid * STRIPE, STRIPE)], zb_v)
        pltpu.sync_copy(zb_v, out_hbm.at[cid].at[pl.ds(sid * STRIPE, STRIPE)])

    return scatter_k(eout_p, col2d)


def _edge_body(gs_ref, gt_ref, ef_ref, A_ref, Bm_ref, Wef_ref, wsq_ref,
               wdist_ref, eb1_ref, eg1_ref, ebt1_ref, eW2_ref, eb2_ref,
               eg2_ref, ebt2_ref, out_ref):
    s = gs_ref[...]
    t = gt_ref[...]
    diff = s - t
    sq = jnp.sum(diff * diff, axis=1, keepdims=True)
    dist = jnp.sqrt(sq + 1e-12)
    h = (jnp.dot(s, A_ref[...], preferred_element_type=jnp.float32)
         + jnp.dot(t, Bm_ref[...], preferred_element_type=jnp.float32)
         + jnp.dot(ef_ref[...], Wef_ref[...], preferred_element_type=jnp.float32)
         + sq * wsq_ref[...] + dist * wdist_ref[...] + eb1_ref[...])
    h = _ln_gelu(h, eg1_ref[...], ebt1_ref[...])
    h2 = jnp.dot(h, eW2_ref[...], preferred_element_type=jnp.float32) + eb2_ref[...]
    h2 = _ln_gelu(h2, eg2_ref[...], ebt2_ref[...])
    gid = pl.program_id(0) * BE + lax.broadcasted_iota(jnp.int32, (BE, 1), 0)
    out_ref[...] = jnp.where(gid < E, h2, 0.0)


def _tc_edge(gs, gt, ef_p, A, Bm, Wef, wsq, wdist, eb1, eg1, ebt1,
             eW2, eb2, eg2, ebt2):
    full = lambda shape: pl.BlockSpec(shape, lambda i: (0, 0))
    return pl.pallas_call(
        _edge_body,
        grid=(EPAD // BE,),
        in_specs=[
            pl.BlockSpec((BE, D), lambda i: (i, 0)),
            pl.BlockSpec((BE, D), lambda i: (i, 0)),
            pl.BlockSpec((BE, DE), lambda i: (i, 0)),
            full((D, EH)), full((D, EH)), full((DE, EH)),
            full((1, EH)), full((1, EH)), full((1, EH)), full((1, EH)),
            full((1, EH)),
            full((EH, OE)), full((1, OE)), full((1, OE)), full((1, OE)),
        ],
        out_specs=pl.BlockSpec((BE, OE), lambda i: (i, 0)),
        out_shape=jax.ShapeDtypeStruct((EPAD, OE), jnp.float32),
    )(gs, gt, ef_p, A, Bm, Wef, wsq, wdist, eb1, eg1, ebt1,
      eW2, eb2, eg2, ebt2)


def _node_body(x_ref, ap_ref, W1x_ref, W1a_ref, nb1_ref, ng1_ref, nbt1_ref,
               nW2_ref, nb2_ref, ng2_ref, nbt2_ref, out_ref):
    x = x_ref[...]
    a = ap_ref[0] + ap_ref[1]
    h = (jnp.dot(x, W1x_ref[...], preferred_element_type=jnp.float32)
         + jnp.dot(a, W1a_ref[...], preferred_element_type=jnp.float32)
         + nb1_ref[...])
    h = _ln_gelu(h, ng1_ref[...], nbt1_ref[...])
    o = jnp.dot(h, nW2_ref[...], preferred_element_type=jnp.float32) + nb2_ref[...]
    out_ref[...] = _ln_gelu(o, ng2_ref[...], nbt2_ref[...])


def _tc_node(x, ap, W1x, W1a, nb1, ng1, nbt1, nW2, nb2, ng2, nbt2):
    full = lambda shape: pl.BlockSpec(shape, lambda i: (0, 0))
    full3 = lambda shape: pl.BlockSpec(shape, lambda i: (0, i, 0))
    return pl.pallas_call(
        _node_body,
        grid=(N // BN,),
        in_specs=[
            pl.BlockSpec((BN, D), lambda i: (i, 0)),
            full3((NC, BN, OE)),
            full((D, NH)), full((OE, NH)),
            full((1, NH)), full((1, NH)), full((1, NH)),
            full((NH, ON)), full((1, ON)), full((1, ON)), full((1, ON)),
        ],
        out_specs=pl.BlockSpec((BN, ON), lambda i: (i, 0)),
        out_shape=jax.ShapeDtypeStruct((N, ON), jnp.float32),
    )(x, ap, W1x, W1a, nb1, ng1, nbt1, nW2, nb2, ng2, nbt2)


def kernel(node_features, edge_index, edge_features,
           eW1, eb1, eg1, ebt1, eW2, eb2, eg2, ebt2,
           nW1, nb1, ng1, nbt1, nW2, nb2, ng2, nbt2):
    row = edge_index[0].astype(jnp.int32)
    col = edge_index[1].astype(jnp.int32)
    row2d = jnp.pad(row, (0, EPAD - E)).reshape(EPAD // 128, 128)
    col2d = jnp.pad(col, (0, EPAD - E)).reshape(EPAD // 128, 128)
    ef_p = jnp.pad(edge_features, ((0, EPAD - E), (0, 0)))

    A = eW1[0:D] + eW1[2 * D:3 * D]
    Bm = eW1[D:2 * D] - eW1[2 * D:3 * D]
    wsq = eW1[3 * D:3 * D + 1]
    wdist = eW1[3 * D + 1:3 * D + 2]
    Wef = eW1[3 * D + 2:]

    r2 = lambda v: v.reshape(1, -1)

    gs, gt = _sc_gather(node_features, row2d, col2d)
    eout_p = _tc_edge(gs, gt, ef_p, A, Bm, Wef, wsq, wdist,
                      r2(eb1), r2(eg1), r2(ebt1), eW2, r2(eb2), r2(eg2),
                      r2(ebt2))
    e_out = eout_p[:E]

    partials = _sc_segment_sum(eout_p, col2d)

    W1x = nW1[0:D]
    W1a = nW1[D:]
    n_out = _tc_node(node_features, partials, W1x, W1a,
                     r2(nb1), r2(ng1), r2(nbt1),
                     nW2, r2(nb2), r2(ng2), r2(nbt2))
    return (n_out, e_out)
